# Initial kernel scaffold; baseline (speedup 1.0000x reference)
import jax
import jax.numpy as jnp
from jax.experimental import pallas as pl


def kernel(h, edge_index, edge_attr, W_e1, b_e1, W_e2, b_e2, W_inf, b_inf, W_n1, b_n1, W_n2, b_n2):
    raise NotImplementedError("write your pallas kernel here")


# speedup vs baseline: 2.4016x; 2.4016x over previous
import functools

import jax
import jax.numpy as jnp
from jax import lax
from jax.experimental import pallas as pl
from jax.experimental.pallas import tpu as pltpu
from jax.experimental.pallas import tpu_sc as plsc

N = 10000
E = 320000
H = 128
EF = 16

NC = 2
NS = 16
NW = NC * NS


_BN = 1000


def _proj_body(h_ref, wi_ref, wj_ref, g_ref, k_ref):
    hb = h_ref[...]
    g_ref[...] = jnp.dot(hb, wi_ref[...], preferred_element_type=jnp.float32)
    k_ref[...] = jnp.dot(hb, wj_ref[...], preferred_element_type=jnp.float32)


def _proj(h, wi, wj):
    grid = (N // _BN,)
    return pl.pallas_call(
        _proj_body,
        grid=grid,
        in_specs=[
            pl.BlockSpec((_BN, H), lambda i: (i, 0)),
            pl.BlockSpec((H, H), lambda i: (0, 0)),
            pl.BlockSpec((H, H), lambda i: (0, 0)),
        ],
        out_specs=[
            pl.BlockSpec((_BN, H), lambda i: (i, 0)),
            pl.BlockSpec((_BN, H), lambda i: (i, 0)),
        ],
        out_shape=[
            jax.ShapeDtypeStruct((N, H), jnp.float32),
            jax.ShapeDtypeStruct((N, H), jnp.float32),
        ],
    )(h, wi, wj)


_GC = 80
_PER_W = E // NW
_GCHUNKS = _PER_W // _GC

@functools.cache
def _sc_mesh():
    return plsc.VectorSubcoreMesh(core_axis_name="c", subcore_axis_name="s",
                                  num_cores=NC, num_subcores=NS)


@functools.cache
def _gather_call():
    return pl.kernel(
        _gather_body,
        out_type=[
            jax.ShapeDtypeStruct((E, H), jnp.float32),
            jax.ShapeDtypeStruct((E, H), jnp.float32),
        ],
        mesh=_sc_mesh(),
        scratch_types=[
            pltpu.VMEM((_GC,), jnp.int32),
            pltpu.VMEM((_GC,), jnp.int32),
            pltpu.VMEM((_GC, H), jnp.float32),
            pltpu.VMEM((_GC, H), jnp.float32),
            pltpu.SemaphoreType.DMA,
            pltpu.SemaphoreType.DMA,
        ],
    )


def _gather_body(g_hbm, k_hbm, dst_hbm, src_hbm, hig_hbm, hjk_hbm,
                 didx, sidx, bufg, bufk, sem1, sem2):
    wid = lax.axis_index("s") * NC + lax.axis_index("c")
    base0 = wid * _PER_W

    def chunk(i, carry):
        base = base0 + i * _GC
        pltpu.sync_copy(dst_hbm.at[pl.ds(base, _GC)], didx)
        pltpu.sync_copy(src_hbm.at[pl.ds(base, _GC)], sidx)
        cg = pltpu.async_copy(g_hbm.at[didx], bufg, sem1)
        ck = pltpu.async_copy(k_hbm.at[sidx], bufk, sem2)
        cg.wait()
        ck.wait()
        pltpu.sync_copy(bufg, hig_hbm.at[pl.ds(base, _GC)])
        pltpu.sync_copy(bufk, hjk_hbm.at[pl.ds(base, _GC)])
        return carry

    lax.fori_loop(0, _GCHUNKS, chunk, 0)


_BE = 512


def _edge_body(hig_ref, hjk_ref, ea_ref, wa_ref, be1_ref, we2_ref, be2_ref,
               winf_ref, binf_ref, w_ref):
    x1 = (hig_ref[...] + hjk_ref[...]
          + jnp.dot(ea_ref[...], wa_ref[...], preferred_element_type=jnp.float32)
          + be1_ref[...])
    x1 = jnp.maximum(x1, 0.0)
    m = jnp.dot(x1, we2_ref[...], preferred_element_type=jnp.float32) + be2_ref[...]
    m = jnp.maximum(m, 0.0)
    logit = jnp.sum(m * winf_ref[...], axis=1, keepdims=True) + binf_ref[...]
    s = jax.nn.sigmoid(logit)
    w_ref[...] = m * s


def _edge_mlp(hig, hjk, ea, wa, be1, we2, be2, winf_row, binf):
    grid = (E // _BE,)
    return pl.pallas_call(
        _edge_body,
        grid=grid,
        in_specs=[
            pl.BlockSpec((_BE, H), lambda i: (i, 0)),
            pl.BlockSpec((_BE, H), lambda i: (i, 0)),
            pl.BlockSpec((_BE, EF), lambda i: (i, 0)),
            pl.BlockSpec((EF, H), lambda i: (0, 0)),
            pl.BlockSpec((1, H), lambda i: (0, 0)),
            pl.BlockSpec((H, H), lambda i: (0, 0)),
            pl.BlockSpec((1, H), lambda i: (0, 0)),
            pl.BlockSpec((1, H), lambda i: (0, 0)),
            pl.BlockSpec((1, 1), lambda i: (0, 0)),
        ],
        out_specs=pl.BlockSpec((_BE, H), lambda i: (i, 0)),
        out_shape=jax.ShapeDtypeStruct((E, H), jnp.float32),
    )(hig, hjk, ea, wa, be1, we2, be2, winf_row, binf)


_SCC = 80
_PER_T = E // NC // NS
_SCHUNKS = _PER_T // _SCC
_NPAD = 10240
_NPT = _NPAD // NS
_ZROWS = 128


@functools.cache
def _scatter_call():
    return pl.kernel(
        _scatter_body,
        out_type=jax.ShapeDtypeStruct((NC, _NPAD, H), jnp.float32),
        mesh=_sc_mesh(),
        scratch_types=[
            pltpu.VMEM((_SCHUNKS, _SCC), jnp.int32),
            pltpu.VMEM((_SCC, H), jnp.float32),
            pltpu.VMEM((_ZROWS, H), jnp.float32),
            pltpu.VMEM_SHARED((_NPAD, H), jnp.float32),
        ],
    )


def _scatter_body(w_hbm, dst_hbm, out_hbm, idx2d, wbuf, zbuf, acc):
    c = lax.axis_index("c")
    s = lax.axis_index("s")

    def zrow(r, carry):
        for cc in range(H // 16):
            zbuf[r, pl.ds(cc * 16, 16)] = jnp.zeros((16,), jnp.float32)
        return carry

    lax.fori_loop(0, _ZROWS, zrow, 0)

    def zcopy(z, carry):
        pltpu.sync_copy(zbuf, acc.at[pl.ds(s * _NPT + z * _ZROWS, _ZROWS)])
        return carry

    lax.fori_loop(0, _NPT // _ZROWS, zcopy, 0)
    plsc.subcore_barrier()

    base0 = c * (E // NC) + s * _PER_T

    def chunk(i, carry):
        base = base0 + i * _SCC
        pltpu.sync_copy(dst_hbm.at[pl.ds(base, _SCC)], idx2d.at[i])
        pltpu.sync_copy(w_hbm.at[pl.ds(base, _SCC)], wbuf)
        pltpu.sync_copy(wbuf, acc.at[idx2d.at[i]], add=True)
        return carry

    lax.fori_loop(0, _SCHUNKS, chunk, 0)
    plsc.subcore_barrier()

    pltpu.sync_copy(acc.at[pl.ds(s * _NPT, _NPT)],
                    out_hbm.at[c, pl.ds(s * _NPT, _NPT)])


def _node_body(mi0_ref, mi1_ref, h_ref, wm_ref, wh_ref, bn1_ref, wn2_ref,
               bn2_ref, out_ref):
    mi = mi0_ref[...] + mi1_ref[...]
    z1 = (jnp.dot(mi, wm_ref[...], preferred_element_type=jnp.float32)
          + jnp.dot(h_ref[...], wh_ref[...], preferred_element_type=jnp.float32)
          + bn1_ref[...])
    z1 = jnp.maximum(z1, 0.0)
    out_ref[...] = (jnp.dot(z1, wn2_ref[...], preferred_element_type=jnp.float32)
                    + bn2_ref[...])


def _node_mlp(mi0, mi1, h, wm, wh, bn1, wn2, bn2):
    grid = (N // _BN,)
    return pl.pallas_call(
        _node_body,
        grid=grid,
        in_specs=[
            pl.BlockSpec((_BN, H), lambda i: (i, 0)),
            pl.BlockSpec((_BN, H), lambda i: (i, 0)),
            pl.BlockSpec((_BN, H), lambda i: (i, 0)),
            pl.BlockSpec((H, H), lambda i: (0, 0)),
            pl.BlockSpec((H, H), lambda i: (0, 0)),
            pl.BlockSpec((1, H), lambda i: (0, 0)),
            pl.BlockSpec((H, H), lambda i: (0, 0)),
            pl.BlockSpec((1, H), lambda i: (0, 0)),
        ],
        out_specs=pl.BlockSpec((_BN, H), lambda i: (i, 0)),
        out_shape=jax.ShapeDtypeStruct((N, H), jnp.float32),
    )(mi0, mi1, h, wm, wh, bn1, wn2, bn2)


def kernel(h, edge_index, edge_attr, W_e1, b_e1, W_e2, b_e2, W_inf, b_inf,
           W_n1, b_n1, W_n2, b_n2):
    dst = edge_index[0].astype(jnp.int32)
    src = edge_index[1].astype(jnp.int32)

    g, k = _proj(h, W_e1[EF:EF + H], W_e1[EF + H:])
    hig, hjk = _gather_call()(g, k, dst, src)
    w = _edge_mlp(hig, hjk, edge_attr, W_e1[:EF],
                  b_e1.reshape(1, H), W_e2, b_e2.reshape(1, H),
                  W_inf.reshape(1, H), b_inf.reshape(1, 1))
    mi2 = _scatter_call()(w, dst)
    out = _node_mlp(mi2[0], mi2[1], h, W_n1[:H], W_n1[H:],
                    b_n1.reshape(1, H), W_n2, b_n2.reshape(1, H))
    return out


# speedup vs baseline: 2.7085x; 1.1278x over previous
import functools

import jax
import jax.numpy as jnp
from jax import lax
from jax.experimental import pallas as pl
from jax.experimental.pallas import tpu as pltpu
from jax.experimental.pallas import tpu_sc as plsc

N = 10000
E = 320000
H = 128
EF = 16

NC = 2
NS = 16
NW = NC * NS


_BN = 1000


def _proj_body(h_ref, wi_ref, wj_ref, g_ref, k_ref):
    hb = h_ref[...]
    g_ref[...] = jnp.dot(hb, wi_ref[...], preferred_element_type=jnp.float32)
    k_ref[...] = jnp.dot(hb, wj_ref[...], preferred_element_type=jnp.float32)


def _proj(h, wi, wj):
    grid = (N // _BN,)
    return pl.pallas_call(
        _proj_body,
        grid=grid,
        in_specs=[
            pl.BlockSpec((_BN, H), lambda i: (i, 0)),
            pl.BlockSpec((H, H), lambda i: (0, 0)),
            pl.BlockSpec((H, H), lambda i: (0, 0)),
        ],
        out_specs=[
            pl.BlockSpec((_BN, H), lambda i: (i, 0)),
            pl.BlockSpec((_BN, H), lambda i: (i, 0)),
        ],
        out_shape=[
            jax.ShapeDtypeStruct((N, H), jnp.float32),
            jax.ShapeDtypeStruct((N, H), jnp.float32),
        ],
    )(h, wi, wj)


_GC = 80
_PER_W = E // NW
_GCHUNKS = _PER_W // _GC

@functools.cache
def _sc_mesh():
    return plsc.VectorSubcoreMesh(core_axis_name="c", subcore_axis_name="s",
                                  num_cores=NC, num_subcores=NS)


@functools.cache
def _gather_call():
    return pl.kernel(
        _gather_body,
        out_type=jax.ShapeDtypeStruct((E, H), jnp.float32),
        mesh=_sc_mesh(),
        scratch_types=[
            pltpu.VMEM((2, _GC), jnp.int32),
            pltpu.VMEM((2, _GC), jnp.int32),
            pltpu.VMEM((2, _GC, H), jnp.float32),
            pltpu.VMEM((2, _GC, H), jnp.float32),
            pltpu.SemaphoreType.DMA,
            pltpu.SemaphoreType.DMA,
            pltpu.SemaphoreType.DMA,
        ],
    )


def _gather_body(g_hbm, k_hbm, dst_hbm, src_hbm, u_hbm,
                 didx, sidx, bufg, bufk, isem, gsem, wsem):
    wid = lax.axis_index("s") * NC + lax.axis_index("c")
    base0 = wid * _PER_W

    def idx_start(i, slot):
        base = base0 + i * _GC
        pltpu.async_copy(dst_hbm.at[pl.ds(base, _GC)], didx.at[slot], isem)
        pltpu.async_copy(src_hbm.at[pl.ds(base, _GC)], sidx.at[slot], isem)

    def idx_wait(i, slot):
        base = base0 + i * _GC
        pltpu.make_async_copy(dst_hbm.at[pl.ds(base, _GC)], didx.at[slot],
                              isem).wait()
        pltpu.make_async_copy(src_hbm.at[pl.ds(base, _GC)], sidx.at[slot],
                              isem).wait()

    def g_start(slot):
        pltpu.async_copy(g_hbm.at[didx.at[slot]], bufg.at[slot], gsem)
        pltpu.async_copy(k_hbm.at[sidx.at[slot]], bufk.at[slot], gsem)

    def g_wait(slot):
        pltpu.make_async_copy(g_hbm.at[didx.at[slot]], bufg.at[slot],
                              gsem).wait()
        pltpu.make_async_copy(k_hbm.at[sidx.at[slot]], bufk.at[slot],
                              gsem).wait()

    def w_start(i, slot):
        base = base0 + i * _GC
        pltpu.async_copy(bufg.at[slot], u_hbm.at[pl.ds(base, _GC)], wsem)

    def w_wait(i, slot):
        base = base0 + i * _GC
        pltpu.make_async_copy(bufg.at[slot], u_hbm.at[pl.ds(base, _GC)],
                              wsem).wait()

    idx_start(0, 0)
    idx_wait(0, 0)
    g_start(0)
    idx_start(1, 1)

    def step(i, carry):
        a = lax.rem(i, 2)
        b = 1 - a
        g_wait(a)

        @pl.when(i < _GCHUNKS - 1)
        def _():
            idx_wait(i + 1, b)

            @pl.when(i >= 1)
            def _():
                w_wait(i - 1, b)

            g_start(b)

            @pl.when(i + 2 < _GCHUNKS)
            def _():
                idx_start(i + 2, a)

        def row(r, c2):
            for cc in range(H // 16):
                sl = pl.ds(cc * 16, 16)
                bufg[a, r, sl] = bufg[a, r, sl] + bufk[a, r, sl]
            return c2

        lax.fori_loop(0, _GC, row, 0, unroll=4)
        w_start(i, a)
        return carry

    lax.fori_loop(0, _GCHUNKS, step, 0)
    w_wait(_GCHUNKS - 2, (_GCHUNKS - 2) % 2)
    w_wait(_GCHUNKS - 1, (_GCHUNKS - 1) % 2)


_BE = 512


def _edge_body(u_ref, ea_ref, wa_ref, be1_ref, we2_ref, be2_ref,
               winf_ref, binf_ref, w_ref):
    x1 = (u_ref[...]
          + jnp.dot(ea_ref[...], wa_ref[...], preferred_element_type=jnp.float32)
          + be1_ref[...])
    x1 = jnp.maximum(x1, 0.0)
    m = jnp.dot(x1, we2_ref[...], preferred_element_type=jnp.float32) + be2_ref[...]
    m = jnp.maximum(m, 0.0)
    logit = jnp.sum(m * winf_ref[...], axis=1, keepdims=True) + binf_ref[...]
    s = jax.nn.sigmoid(logit)
    w_ref[...] = m * s


def _edge_mlp(u, ea, wa, be1, we2, be2, winf_row, binf):
    grid = (E // _BE,)
    return pl.pallas_call(
        _edge_body,
        grid=grid,
        in_specs=[
            pl.BlockSpec((_BE, H), lambda i: (i, 0)),
            pl.BlockSpec((_BE, EF), lambda i: (i, 0)),
            pl.BlockSpec((EF, H), lambda i: (0, 0)),
            pl.BlockSpec((1, H), lambda i: (0, 0)),
            pl.BlockSpec((H, H), lambda i: (0, 0)),
            pl.BlockSpec((1, H), lambda i: (0, 0)),
            pl.BlockSpec((1, H), lambda i: (0, 0)),
            pl.BlockSpec((1, 1), lambda i: (0, 0)),
        ],
        out_specs=pl.BlockSpec((_BE, H), lambda i: (i, 0)),
        out_shape=jax.ShapeDtypeStruct((E, H), jnp.float32),
    )(u, ea, wa, be1, we2, be2, winf_row, binf)


_SCC = 80
_PER_T = E // NC // NS
_SCHUNKS = _PER_T // _SCC
_NPAD = 10240
_NPT = _NPAD // NS
_ZROWS = 128


@functools.cache
def _scatter_call():
    return pl.kernel(
        _scatter_body,
        out_type=jax.ShapeDtypeStruct((NC, _NPAD, H), jnp.float32),
        mesh=_sc_mesh(),
        scratch_types=[
            pltpu.VMEM((2, _SCC), jnp.int32),
            pltpu.VMEM((2, _SCC, H), jnp.float32),
            pltpu.VMEM((_ZROWS, H), jnp.float32),
            pltpu.VMEM_SHARED((_NPAD, H), jnp.float32),
            pltpu.SemaphoreType.DMA,
            pltpu.SemaphoreType.DMA,
        ],
    )


def _scatter_body(w_hbm, dst_hbm, out_hbm, idx2d, wbuf, zbuf, acc, lsem, ssem):
    c = lax.axis_index("c")
    s = lax.axis_index("s")

    def zrow(r, carry):
        for cc in range(H // 16):
            zbuf[r, pl.ds(cc * 16, 16)] = jnp.zeros((16,), jnp.float32)
        return carry

    lax.fori_loop(0, _ZROWS, zrow, 0)

    def zcopy(z, carry):
        pltpu.sync_copy(zbuf, acc.at[pl.ds(s * _NPT + z * _ZROWS, _ZROWS)])
        return carry

    lax.fori_loop(0, _NPT // _ZROWS, zcopy, 0)
    plsc.subcore_barrier()

    base0 = c * (E // NC) + s * _PER_T

    def l_start(i, slot):
        base = base0 + i * _SCC
        pltpu.async_copy(dst_hbm.at[pl.ds(base, _SCC)], idx2d.at[slot], lsem)
        pltpu.async_copy(w_hbm.at[pl.ds(base, _SCC)], wbuf.at[slot], lsem)

    def l_wait(i, slot):
        base = base0 + i * _SCC
        pltpu.make_async_copy(dst_hbm.at[pl.ds(base, _SCC)], idx2d.at[slot],
                              lsem).wait()
        pltpu.make_async_copy(w_hbm.at[pl.ds(base, _SCC)], wbuf.at[slot],
                              lsem).wait()

    def sc_start(slot):
        pltpu.async_copy(wbuf.at[slot], acc.at[idx2d.at[slot]], ssem, add=True)

    def sc_wait(slot):
        pltpu.make_async_copy(wbuf.at[slot], acc.at[idx2d.at[slot]],
                              ssem).wait()

    l_start(0, 0)

    def chunk(i, carry):
        a = lax.rem(i, 2)
        b = 1 - a
        l_wait(i, a)

        @pl.when(i >= 1)
        def _():
            sc_wait(b)

        @pl.when(i < _SCHUNKS - 1)
        def _():
            l_start(i + 1, b)

        sc_start(a)
        return carry

    lax.fori_loop(0, _SCHUNKS, chunk, 0)
    sc_wait((_SCHUNKS - 1) % 2)
    plsc.subcore_barrier()

    pltpu.sync_copy(acc.at[pl.ds(s * _NPT, _NPT)],
                    out_hbm.at[c, pl.ds(s * _NPT, _NPT)])


def _node_body(mi0_ref, mi1_ref, h_ref, wm_ref, wh_ref, bn1_ref, wn2_ref,
               bn2_ref, out_ref):
    mi = mi0_ref[...] + mi1_ref[...]
    z1 = (jnp.dot(mi, wm_ref[...], preferred_element_type=jnp.float32)
          + jnp.dot(h_ref[...], wh_ref[...], preferred_element_type=jnp.float32)
          + bn1_ref[...])
    z1 = jnp.maximum(z1, 0.0)
    out_ref[...] = (jnp.dot(z1, wn2_ref[...], preferred_element_type=jnp.float32)
                    + bn2_ref[...])


def _node_mlp(mi0, mi1, h, wm, wh, bn1, wn2, bn2):
    grid = (N // _BN,)
    return pl.pallas_call(
        _node_body,
        grid=grid,
        in_specs=[
            pl.BlockSpec((_BN, H), lambda i: (i, 0)),
            pl.BlockSpec((_BN, H), lambda i: (i, 0)),
            pl.BlockSpec((_BN, H), lambda i: (i, 0)),
            pl.BlockSpec((H, H), lambda i: (0, 0)),
            pl.BlockSpec((H, H), lambda i: (0, 0)),
            pl.BlockSpec((1, H), lambda i: (0, 0)),
            pl.BlockSpec((H, H), lambda i: (0, 0)),
            pl.BlockSpec((1, H), lambda i: (0, 0)),
        ],
        out_specs=pl.BlockSpec((_BN, H), lambda i: (i, 0)),
        out_shape=jax.ShapeDtypeStruct((N, H), jnp.float32),
    )(mi0, mi1, h, wm, wh, bn1, wn2, bn2)


def kernel(h, edge_index, edge_attr, W_e1, b_e1, W_e2, b_e2, W_inf, b_inf,
           W_n1, b_n1, W_n2, b_n2):
    dst = edge_index[0].astype(jnp.int32)
    src = edge_index[1].astype(jnp.int32)

    g, k = _proj(h, W_e1[EF:EF + H], W_e1[EF + H:])
    u = _gather_call()(g, k, dst, src)
    w = _edge_mlp(u, edge_attr, W_e1[:EF],
                  b_e1.reshape(1, H), W_e2, b_e2.reshape(1, H),
                  W_inf.reshape(1, H), b_inf.reshape(1, 1))
    mi2 = _scatter_call()(w, dst)
    out = _node_mlp(mi2[0], mi2[1], h, W_n1[:H], W_n1[H:],
                    b_n1.reshape(1, H), W_n2, b_n2.reshape(1, H))
    return out
